# baseline (speedup 1.0000x reference)
import functools

import jax
import jax.numpy as jnp
from jax import lax
from jax.experimental import pallas as pl
from jax.experimental.pallas import tpu as pltpu
from jax.experimental.pallas import tpu_sc as plsc

D = 64
FINAL = 192
LANES = 16
NC = 2
NS = 16
ECHUNK = 80
FCH = 80


def _leaky(x):
    return jnp.where(x >= 0, x, 0.01 * x)


@functools.cache
def _make_spmm(n_nodes: int, n_edges: int):
    half = n_nodes // 2
    assert half * 2 == n_nodes
    e_per_tile = n_edges // NS
    G = e_per_tile // ECHUNK
    assert G * ECHUNK == e_per_tile and G >= 4
    half_pad = -(-(half + 8) // (NS * 8)) * (NS * 8)
    zrows = half_pad // NS
    wb = -(-half // NS)
    wb = -(-wb // 8) * 8
    wb_last = half - wb * (NS - 1)
    assert wb_last > 0 and wb_last % 8 == 0

    mesh = plsc.VectorSubcoreMesh(core_axis_name="c", subcore_axis_name="s")

    @functools.partial(
        pl.kernel,
        out_type=jax.ShapeDtypeStruct((n_nodes, D), jnp.float32),
        mesh=mesh,
        scratch_types=[
            pltpu.VMEM((2, ECHUNK), jnp.int32),
            pltpu.VMEM((2, ECHUNK), jnp.int32),
            pltpu.VMEM((2, ECHUNK), jnp.float32),
            pltpu.VMEM((2, ECHUNK), jnp.int32),
            pltpu.VMEM((2, ECHUNK, D), jnp.float32),
            pltpu.VMEM_SHARED((half_pad, D), jnp.float32),
            pltpu.SemaphoreType.DMA,
            pltpu.SemaphoreType.DMA,
            pltpu.SemaphoreType.DMA,
            pltpu.SemaphoreType.DMA,
            pltpu.SemaphoreType.DMA,
        ],
        compiler_params=pltpu.CompilerParams(use_tc_tiling_on_sc=False),
    )
    def spmm(ego_hbm, src_hbm, dst_hbm, vals_hbm, zeros_hbm, out_hbm,
             src_v, dst_v, vals_v, sidx_v, rows_v, acc, isem, gs0, gs1,
             ss0, ss1):
        cid = lax.axis_index("c")
        sid = lax.axis_index("s")
        off = cid * half
        gsem = (gs0, gs1)
        ssem = (ss0, ss1)

        pltpu.sync_copy(zeros_hbm.at[pl.ds(sid * zrows, zrows)],
                        acc.at[pl.ds(sid * zrows, zrows)])
        plsc.subcore_barrier()

        ebase = sid * e_per_tile

        def fire_idx(gg, b):
            eb = ebase + gg * ECHUNK
            pltpu.async_copy(src_hbm.at[pl.ds(eb, ECHUNK)], src_v.at[b], isem)
            pltpu.async_copy(dst_hbm.at[pl.ds(eb, ECHUNK)], dst_v.at[b], isem)
            pltpu.async_copy(vals_hbm.at[pl.ds(eb, ECHUNK)], vals_v.at[b], isem)

        def wait_idx(b):
            pltpu.make_async_copy(src_hbm.at[pl.ds(0, ECHUNK)], src_v.at[b],
                                  isem).wait()
            pltpu.make_async_copy(dst_hbm.at[pl.ds(0, ECHUNK)], dst_v.at[b],
                                  isem).wait()
            pltpu.make_async_copy(vals_hbm.at[pl.ds(0, ECHUNK)], vals_v.at[b],
                                  isem).wait()

        def fire_gather(b):
            pltpu.async_copy(ego_hbm.at[src_v.at[b]], rows_v.at[b], gsem[b])

        def wait_gather(b):
            pltpu.make_async_copy(ego_hbm.at[src_v.at[b]], rows_v.at[b],
                                  gsem[b]).wait()

        def fire_scatter(b):
            pltpu.async_copy(rows_v.at[b], acc.at[sidx_v.at[b]], ssem[b],
                             add=True)

        def wait_scatter(b):
            pltpu.make_async_copy(rows_v.at[b], acc.at[sidx_v.at[b]],
                                  ssem[b]).wait()

        def compute(b):
            for q in range(ECHUNK // LANES):
                sl = pl.ds(q * LANES, LANES)
                dvec = dst_v[b, sl]
                inr = (dvec >= off) & (dvec < off + half)
                sidx_v[b, sl] = jnp.where(inr, dvec - off, half)
                vvec = vals_v[b, sl]
                for j in range(LANES):
                    e = q * LANES + j
                    v = vvec[j]
                    for c in range(D // LANES):
                        cl = pl.ds(c * LANES, LANES)
                        rows_v[b, e, cl] = rows_v[b, e, cl] * v

        fire_idx(0, 0)
        wait_idx(0)
        fire_gather(0)
        fire_idx(1, 1)

        npairs = G // 2

        @pl.loop(0, npairs)
        def _pair(h):
            for b in (0, 1):
                gg = 2 * h + b
                nb = 1 - b

                @pl.when(gg + 1 < G)
                def _():
                    wait_idx(nb)

                @pl.when(gg >= 1)
                def _():
                    wait_scatter(nb)

                @pl.when(gg + 1 < G)
                def _():
                    fire_gather(nb)

                wait_gather(b)
                compute(b)
                fire_scatter(b)

                @pl.when(gg + 2 < G)
                def _():
                    fire_idx(gg + 2, b)

        if G % 2:
            wait_scatter(1)
            wait_gather(0)
            compute(0)
            fire_scatter(0)
            wait_scatter(0)
        else:
            wait_scatter(1)

        plsc.subcore_barrier()

        @pl.when(sid < NS - 1)
        def _wb_main():
            pltpu.sync_copy(acc.at[pl.ds(sid * wb, wb)],
                            out_hbm.at[pl.ds(off + sid * wb, wb)])

        @pl.when(sid == NS - 1)
        def _wb_tail():
            pltpu.sync_copy(acc.at[pl.ds((NS - 1) * wb, wb_last)],
                            out_hbm.at[pl.ds(off + (NS - 1) * wb, wb_last)])

    return spmm, half_pad


@functools.cache
def _make_dense(n_nodes: int, bs: int):
    grid = n_nodes // bs
    assert grid * bs == n_nodes

    def body(side_ref, ego_ref, wg_ref, bg_ref, wb_ref, bb_ref,
             enew_ref, enorm_ref):
        side = side_ref[...]
        ego = ego_ref[...]
        s = _leaky(jnp.dot(side, wg_ref[...],
                           preferred_element_type=jnp.float32) + bg_ref[...])
        b = _leaky(jnp.dot(ego * side, wb_ref[...],
                           preferred_element_type=jnp.float32) + bb_ref[...])
        e = s + b
        nrm = jnp.maximum(jnp.sqrt(jnp.sum(e * e, axis=1, keepdims=True)), 1e-12)
        enew_ref[...] = e
        enorm_ref[...] = e / nrm

    return pl.pallas_call(
        body,
        grid=(grid,),
        in_specs=[
            pl.BlockSpec((bs, D), lambda i: (i, 0)),
            pl.BlockSpec((bs, D), lambda i: (i, 0)),
            pl.BlockSpec((D, D), lambda i: (0, 0)),
            pl.BlockSpec((1, D), lambda i: (0, 0)),
            pl.BlockSpec((D, D), lambda i: (0, 0)),
            pl.BlockSpec((1, D), lambda i: (0, 0)),
        ],
        out_specs=[pl.BlockSpec((bs, D), lambda i: (i, 0)),
                   pl.BlockSpec((bs, D), lambda i: (i, 0))],
        out_shape=[jax.ShapeDtypeStruct((n_nodes, D), jnp.float32),
                   jax.ShapeDtypeStruct((n_nodes, D), jnp.float32)],
    )


@functools.cache
def _make_sigmoid(n_rows: int, bs: int):
    grid = n_rows // bs
    assert grid * bs == n_rows

    def body(w_ref, r_ref):
        r_ref[...] = jax.nn.sigmoid(w_ref[...])

    return pl.pallas_call(
        body,
        grid=(grid,),
        in_specs=[pl.BlockSpec((bs, FINAL), lambda i: (i, 0))],
        out_specs=pl.BlockSpec((bs, FINAL), lambda i: (i, 0)),
        out_shape=jax.ShapeDtypeStruct((n_rows, FINAL), jnp.float32),
    )


@functools.cache
def _make_fuse(n_pad: int, m_pad: int):
    per_worker = n_pad // (NC * NS)
    G = per_worker // FCH
    assert G * FCH == per_worker and G % 2 == 0 and G >= 4

    mesh = plsc.VectorSubcoreMesh(core_axis_name="c", subcore_axis_name="s")

    @functools.partial(
        pl.kernel,
        out_type=jax.ShapeDtypeStruct((n_pad, FINAL), jnp.float32),
        mesh=mesh,
        scratch_types=[
            pltpu.VMEM((2, FCH), jnp.int32),
            pltpu.VMEM((2, FCH, FINAL), jnp.float32),
            pltpu.VMEM((2, FCH, FINAL), jnp.float32),
            pltpu.VMEM((2, FCH, D), jnp.float32),
            pltpu.VMEM((2, FCH, D), jnp.float32),
            pltpu.VMEM((2, FCH, D), jnp.float32),
            pltpu.SemaphoreType.DMA,
            pltpu.SemaphoreType.DMA,
            pltpu.SemaphoreType.DMA,
            pltpu.SemaphoreType.DMA,
            pltpu.SemaphoreType.DMA,
        ],
        compiler_params=pltpu.CompilerParams(use_tc_tiling_on_sc=False),
    )
    def fuse(base_hbm, wr_hbm, sm0_hbm, sm1_hbm, sm2_hbm, cidx_hbm, out_hbm,
             idx_v, base_v, wr_v, g0_v, g1_v, g2_v, isem, gs0, gs1, ss0, ss1):
        cid = lax.axis_index("c")
        sid = lax.axis_index("s")
        wid = sid * NC + cid
        rbase = wid * per_worker
        gsem = (gs0, gs1)
        ssem = (ss0, ss1)

        def fire_loads(gg, b):
            rb = rbase + gg * FCH
            pltpu.async_copy(cidx_hbm.at[pl.ds(rb, FCH)], idx_v.at[b], isem)
            pltpu.async_copy(base_hbm.at[pl.ds(rb, FCH)], base_v.at[b], isem)
            pltpu.async_copy(wr_hbm.at[pl.ds(rb, FCH)], wr_v.at[b], isem)

        def wait_loads(b):
            pltpu.make_async_copy(cidx_hbm.at[pl.ds(0, FCH)], idx_v.at[b],
                                  isem).wait()
            pltpu.make_async_copy(base_hbm.at[pl.ds(0, FCH)], base_v.at[b],
                                  isem).wait()
            pltpu.make_async_copy(wr_hbm.at[pl.ds(0, FCH)], wr_v.at[b],
                                  isem).wait()

        smalls = (sm0_hbm, sm1_hbm, sm2_hbm)
        gaths = (g0_v, g1_v, g2_v)

        def fire_gather(b):
            for p in range(3):
                pltpu.async_copy(smalls[p].at[idx_v.at[b]], gaths[p].at[b],
                                 gsem[b])

        def wait_gather(b):
            for p in range(3):
                pltpu.make_async_copy(smalls[p].at[idx_v.at[b]],
                                      gaths[p].at[b], gsem[b]).wait()

        def fire_store(gg, b):
            rb = rbase + gg * FCH
            pltpu.async_copy(wr_v.at[b], out_hbm.at[pl.ds(rb, FCH)], ssem[b])

        def wait_store(b):
            pltpu.make_async_copy(wr_v.at[b], out_hbm.at[pl.ds(0, FCH)],
                                  ssem[b]).wait()

        def compute(b):
            def row(r, carry):
                for c in range(FINAL // LANES):
                    sl = pl.ds(c * LANES, LANES)
                    g = gaths[c // 4][b, r, pl.ds((c % 4) * LANES, LANES)]
                    wr_v[b, r, sl] = g + wr_v[b, r, sl] * (base_v[b, r, sl] - g)
                return carry
            lax.fori_loop(0, FCH, row, 0)

        fire_loads(0, 0)
        wait_loads(0)
        fire_gather(0)
        fire_loads(1, 1)

        @pl.loop(0, G // 2)
        def _pair(h):
            for b in (0, 1):
                gg = 2 * h + b
                nb = 1 - b

                @pl.when(gg + 1 < G)
                def _():
                    wait_loads(nb)

                @pl.when(gg >= 1)
                def _():
                    wait_store(nb)

                @pl.when(gg + 1 < G)
                def _():
                    fire_gather(nb)

                wait_gather(b)
                compute(b)
                fire_store(gg, b)

                @pl.when(gg + 2 < G)
                def _():
                    fire_loads(gg + 2, b)

        wait_store(1)

    return fuse


def _ngcf_run(ego, src, dst, vals, Wgc, bgc, Wbi, bbi, bs):
    n = ego.shape[0]
    spmm, half_pad = _make_spmm(n, src.shape[0])
    dense = _make_dense(n, bs)
    zeros = jnp.zeros((half_pad, D), jnp.float32)
    pieces = [ego]
    e = ego
    for l in range(Wgc.shape[0]):
        side = spmm(e, src, dst, vals, zeros)
        e, en = dense(side, e, Wgc[l], bgc[l].reshape(1, D),
                      Wbi[l], bbi[l].reshape(1, D))
        pieces.append(en)
    return pieces


def _fuse_run(base, wr, smalls, idx, sig_bs):
    n, m = base.shape[0], smalls[0].shape[0]
    group = NC * NS * FCH * 2
    n_pad = -(-n // group) * group
    m_pad = m + 8
    fuse = _make_fuse(n_pad, m_pad)

    ratio = _make_sigmoid(n, sig_bs)(wr)

    inv = jnp.full((n,), -1, jnp.int32).at[idx].max(
        jnp.arange(idx.shape[0], dtype=jnp.int32))
    cidx = jnp.where(inv >= 0, inv, m)

    pad_n = n_pad - n
    base_p = jnp.concatenate([base, jnp.zeros((pad_n, FINAL), jnp.float32)])
    wr_p = jnp.concatenate([ratio, jnp.zeros((pad_n, FINAL), jnp.float32)])
    cidx_p = jnp.concatenate([cidx, jnp.full((pad_n,), m, jnp.int32)])
    sm_p = [jnp.concatenate([s, jnp.zeros((m_pad - m, D), jnp.float32)])
            for s in smalls]
    return fuse(base_p, wr_p, sm_p[0], sm_p[1], sm_p[2], cidx_p)[:n]


def kernel(edge_index0, values0, edge_index1, values1, idx_u, idx_i,
           user_emb0, item_emb0, user_emb1, item_emb1,
           Wgc0, bgc0, Wbi0, bbi0, Wgc1, bgc1, Wbi1, bbi1,
           W_ratio_u, W_ratio_i):
    nu0, ni0 = user_emb0.shape[0], item_emb0.shape[0]
    nu1 = user_emb1.shape[0]

    ego0 = jnp.concatenate([user_emb0, item_emb0], axis=0)
    ego1 = jnp.concatenate([user_emb1, item_emb1], axis=0)

    p0 = _ngcf_run(ego0, edge_index0[0], edge_index0[1], values0,
                   Wgc0, bgc0, Wbi0, bbi0, bs=1000)
    p1 = _ngcf_run(ego1, edge_index1[0], edge_index1[1], values1,
                   Wgc1, bgc1, Wbi1, bbi1, bs=1000)
    A0 = jnp.concatenate(p0, axis=1)

    final_u = _fuse_run(A0[:nu0], W_ratio_u, [p[:nu1] for p in p1],
                        idx_u, sig_bs=1000)
    final_i = _fuse_run(A0[nu0:], W_ratio_i, [p[nu1:] for p in p1],
                        idx_i, sig_bs=1000)
    return (final_u, final_i)


# speedup vs baseline: 1.2739x; 1.2739x over previous
import functools

import jax
import jax.numpy as jnp
from jax import lax
from jax.experimental import pallas as pl
from jax.experimental.pallas import tpu as pltpu
from jax.experimental.pallas import tpu_sc as plsc

D = 64
FINAL = 192
LANES = 16
NC = 2
NS = 16
ECHUNK = 80
FCH = 80


def _leaky(x):
    return jnp.where(x >= 0, x, 0.01 * x)


@functools.cache
def _make_spmm(n_nodes: int, n_edges: int):
    half = n_nodes // 2
    assert half * 2 == n_nodes
    e_per_tile = n_edges // NS
    G = e_per_tile // ECHUNK
    assert G * ECHUNK == e_per_tile and G >= 4
    half_pad = -(-(half + 8) // (NS * 8)) * (NS * 8)
    zrows = half_pad // NS
    wb = -(-half // NS)
    wb = -(-wb // 8) * 8
    wb_last = half - wb * (NS - 1)
    assert wb_last > 0 and wb_last % 8 == 0

    mesh = plsc.VectorSubcoreMesh(core_axis_name="c", subcore_axis_name="s")

    @functools.partial(
        pl.kernel,
        out_type=jax.ShapeDtypeStruct((n_nodes, D), jnp.float32),
        mesh=mesh,
        scratch_types=[
            pltpu.VMEM((2, ECHUNK), jnp.int32),
            pltpu.VMEM((2, ECHUNK), jnp.int32),
            pltpu.VMEM((2, ECHUNK), jnp.float32),
            pltpu.VMEM((2, ECHUNK), jnp.int32),
            pltpu.VMEM((2, ECHUNK, D), jnp.float32),
            pltpu.VMEM_SHARED((half_pad, D), jnp.float32),
            pltpu.SemaphoreType.DMA,
            pltpu.SemaphoreType.DMA,
            pltpu.SemaphoreType.DMA,
            pltpu.SemaphoreType.DMA,
            pltpu.SemaphoreType.DMA,
        ],
        compiler_params=pltpu.CompilerParams(use_tc_tiling_on_sc=False),
    )
    def spmm(ego_hbm, src_hbm, dst_hbm, vals_hbm, zeros_hbm, out_hbm,
             src_v, dst_v, vals_v, sidx_v, rows_v, acc, isem, gs0, gs1,
             ss0, ss1):
        cid = lax.axis_index("c")
        sid = lax.axis_index("s")
        off = cid * half
        gsem = (gs0, gs1)
        ssem = (ss0, ss1)

        pltpu.sync_copy(zeros_hbm.at[pl.ds(sid * zrows, zrows)],
                        acc.at[pl.ds(sid * zrows, zrows)])
        plsc.subcore_barrier()

        ebase = sid * e_per_tile

        def fire_idx(gg, b):
            eb = ebase + gg * ECHUNK
            pltpu.async_copy(src_hbm.at[pl.ds(eb, ECHUNK)], src_v.at[b], isem)
            pltpu.async_copy(dst_hbm.at[pl.ds(eb, ECHUNK)], dst_v.at[b], isem)
            pltpu.async_copy(vals_hbm.at[pl.ds(eb, ECHUNK)], vals_v.at[b], isem)

        def wait_idx(b):
            pltpu.make_async_copy(src_hbm.at[pl.ds(0, ECHUNK)], src_v.at[b],
                                  isem).wait()
            pltpu.make_async_copy(dst_hbm.at[pl.ds(0, ECHUNK)], dst_v.at[b],
                                  isem).wait()
            pltpu.make_async_copy(vals_hbm.at[pl.ds(0, ECHUNK)], vals_v.at[b],
                                  isem).wait()

        def fire_gather(b):
            pltpu.async_copy(ego_hbm.at[src_v.at[b]], rows_v.at[b], gsem[b])

        def wait_gather(b):
            pltpu.make_async_copy(ego_hbm.at[src_v.at[b]], rows_v.at[b],
                                  gsem[b]).wait()

        def fire_scatter(b):
            pltpu.async_copy(rows_v.at[b], acc.at[sidx_v.at[b]], ssem[b],
                             add=True)

        def wait_scatter(b):
            pltpu.make_async_copy(rows_v.at[b], acc.at[sidx_v.at[b]],
                                  ssem[b]).wait()

        def compute(b):
            for q in range(ECHUNK // LANES):
                sl = pl.ds(q * LANES, LANES)
                dvec = dst_v[b, sl]
                inr = (dvec >= off) & (dvec < off + half)
                sidx_v[b, sl] = jnp.where(inr, dvec - off, half)
                vvec = vals_v[b, sl]
                for j in range(LANES):
                    e = q * LANES + j
                    v = vvec[j]
                    for c in range(D // LANES):
                        cl = pl.ds(c * LANES, LANES)
                        rows_v[b, e, cl] = rows_v[b, e, cl] * v

        fire_idx(0, 0)
        wait_idx(0)
        fire_gather(0)
        fire_idx(1, 1)

        npairs = G // 2

        @pl.loop(0, npairs)
        def _pair(h):
            for b in (0, 1):
                gg = 2 * h + b
                nb = 1 - b

                @pl.when(gg + 1 < G)
                def _():
                    wait_idx(nb)

                @pl.when(gg >= 1)
                def _():
                    wait_scatter(nb)

                @pl.when(gg + 1 < G)
                def _():
                    fire_gather(nb)

                wait_gather(b)
                compute(b)
                fire_scatter(b)

                @pl.when(gg + 2 < G)
                def _():
                    fire_idx(gg + 2, b)

        if G % 2:
            wait_scatter(1)
            wait_gather(0)
            compute(0)
            fire_scatter(0)
            wait_scatter(0)
        else:
            wait_scatter(1)

        plsc.subcore_barrier()

        @pl.when(sid < NS - 1)
        def _wb_main():
            pltpu.sync_copy(acc.at[pl.ds(sid * wb, wb)],
                            out_hbm.at[pl.ds(off + sid * wb, wb)])

        @pl.when(sid == NS - 1)
        def _wb_tail():
            pltpu.sync_copy(acc.at[pl.ds((NS - 1) * wb, wb_last)],
                            out_hbm.at[pl.ds(off + (NS - 1) * wb, wb_last)])

    return spmm, half_pad


@functools.cache
def _make_dense(n_nodes: int, bs: int):
    grid = n_nodes // bs
    assert grid * bs == n_nodes

    def body(side_ref, ego_ref, wg_ref, bg_ref, wb_ref, bb_ref,
             enew_ref, enorm_ref):
        side = side_ref[...]
        ego = ego_ref[...]
        s = _leaky(jnp.dot(side, wg_ref[...],
                           preferred_element_type=jnp.float32) + bg_ref[...])
        b = _leaky(jnp.dot(ego * side, wb_ref[...],
                           preferred_element_type=jnp.float32) + bb_ref[...])
        e = s + b
        nrm = jnp.maximum(jnp.sqrt(jnp.sum(e * e, axis=1, keepdims=True)), 1e-12)
        enew_ref[...] = e
        enorm_ref[...] = e / nrm

    return pl.pallas_call(
        body,
        grid=(grid,),
        in_specs=[
            pl.BlockSpec((bs, D), lambda i: (i, 0)),
            pl.BlockSpec((bs, D), lambda i: (i, 0)),
            pl.BlockSpec((D, D), lambda i: (0, 0)),
            pl.BlockSpec((1, D), lambda i: (0, 0)),
            pl.BlockSpec((D, D), lambda i: (0, 0)),
            pl.BlockSpec((1, D), lambda i: (0, 0)),
        ],
        out_specs=[pl.BlockSpec((bs, D), lambda i: (i, 0)),
                   pl.BlockSpec((bs, D), lambda i: (i, 0))],
        out_shape=[jax.ShapeDtypeStruct((n_nodes, D), jnp.float32),
                   jax.ShapeDtypeStruct((n_nodes, D), jnp.float32)],
    )


@functools.cache
def _make_sigmoid(n_rows: int, bs: int):
    grid = n_rows // bs
    assert grid * bs == n_rows

    def body(w_ref, r_ref):
        r_ref[...] = jax.nn.sigmoid(w_ref[...])

    return pl.pallas_call(
        body,
        grid=(grid,),
        in_specs=[pl.BlockSpec((bs, FINAL), lambda i: (i, 0))],
        out_specs=pl.BlockSpec((bs, FINAL), lambda i: (i, 0)),
        out_shape=jax.ShapeDtypeStruct((n_rows, FINAL), jnp.float32),
    )


@functools.cache
def _make_fuse(n_pad: int, m_pad: int):
    per_worker = n_pad // (NC * NS)
    G = per_worker // FCH
    assert G * FCH == per_worker and G % 2 == 0 and G >= 4

    mesh = plsc.VectorSubcoreMesh(core_axis_name="c", subcore_axis_name="s")

    @functools.partial(
        pl.kernel,
        out_type=jax.ShapeDtypeStruct((n_pad, FINAL), jnp.float32),
        mesh=mesh,
        scratch_types=[
            pltpu.VMEM((2, FCH), jnp.int32),
            pltpu.VMEM((2, FCH, FINAL), jnp.float32),
            pltpu.VMEM((2, FCH, FINAL), jnp.float32),
            pltpu.VMEM((2, FCH, D), jnp.float32),
            pltpu.VMEM((2, FCH, D), jnp.float32),
            pltpu.VMEM((2, FCH, D), jnp.float32),
            pltpu.SemaphoreType.DMA,
            pltpu.SemaphoreType.DMA,
            pltpu.SemaphoreType.DMA,
            pltpu.SemaphoreType.DMA,
            pltpu.SemaphoreType.DMA,
        ],
        compiler_params=pltpu.CompilerParams(use_tc_tiling_on_sc=False),
    )
    def fuse(base_hbm, wr_hbm, sm0_hbm, sm1_hbm, sm2_hbm, cidx_hbm, out_hbm,
             idx_v, base_v, wr_v, g0_v, g1_v, g2_v, isem, gs0, gs1, ss0, ss1):
        cid = lax.axis_index("c")
        sid = lax.axis_index("s")
        wid = sid * NC + cid
        rbase = wid * per_worker
        gsem = (gs0, gs1)
        ssem = (ss0, ss1)

        def fire_loads(gg, b):
            rb = rbase + gg * FCH
            pltpu.async_copy(cidx_hbm.at[pl.ds(rb, FCH)], idx_v.at[b], isem)
            pltpu.async_copy(base_hbm.at[pl.ds(rb, FCH)], base_v.at[b], isem)
            pltpu.async_copy(wr_hbm.at[pl.ds(rb, FCH)], wr_v.at[b], isem)

        def wait_loads(b):
            pltpu.make_async_copy(cidx_hbm.at[pl.ds(0, FCH)], idx_v.at[b],
                                  isem).wait()
            pltpu.make_async_copy(base_hbm.at[pl.ds(0, FCH)], base_v.at[b],
                                  isem).wait()
            pltpu.make_async_copy(wr_hbm.at[pl.ds(0, FCH)], wr_v.at[b],
                                  isem).wait()

        smalls = (sm0_hbm, sm1_hbm, sm2_hbm)
        gaths = (g0_v, g1_v, g2_v)

        def fire_gather(b):
            for p in range(3):
                pltpu.async_copy(smalls[p].at[idx_v.at[b]], gaths[p].at[b],
                                 gsem[b])

        def wait_gather(b):
            for p in range(3):
                pltpu.make_async_copy(smalls[p].at[idx_v.at[b]],
                                      gaths[p].at[b], gsem[b]).wait()

        def fire_store(gg, b):
            rb = rbase + gg * FCH
            pltpu.async_copy(wr_v.at[b], out_hbm.at[pl.ds(rb, FCH)], ssem[b])

        def wait_store(b):
            pltpu.make_async_copy(wr_v.at[b], out_hbm.at[pl.ds(0, FCH)],
                                  ssem[b]).wait()

        def compute(b):
            def row(r, carry):
                for c in range(FINAL // LANES):
                    sl = pl.ds(c * LANES, LANES)
                    g = gaths[c // 4][b, r, pl.ds((c % 4) * LANES, LANES)]
                    wr_v[b, r, sl] = g + wr_v[b, r, sl] * (base_v[b, r, sl] - g)
                return carry
            lax.fori_loop(0, FCH, row, 0)

        fire_loads(0, 0)
        wait_loads(0)
        fire_gather(0)
        fire_loads(1, 1)

        @pl.loop(0, G // 2)
        def _pair(h):
            for b in (0, 1):
                gg = 2 * h + b
                nb = 1 - b

                @pl.when(gg + 1 < G)
                def _():
                    wait_loads(nb)

                @pl.when(gg >= 1)
                def _():
                    wait_store(nb)

                @pl.when(gg + 1 < G)
                def _():
                    fire_gather(nb)

                wait_gather(b)
                compute(b)
                fire_store(gg, b)

                @pl.when(gg + 2 < G)
                def _():
                    fire_loads(gg + 2, b)

        wait_store(1)

    return fuse


def _ngcf_run(ego, src, dst, vals, Wgc, bgc, Wbi, bbi, bs):
    n = ego.shape[0]
    spmm, half_pad = _make_spmm(n, src.shape[0])
    dense = _make_dense(n, bs)
    zeros = jnp.zeros((half_pad, D), jnp.float32)
    pieces = [ego]
    e = ego
    for l in range(Wgc.shape[0]):
        side = spmm(e, src, dst, vals, zeros)
        e, en = dense(side, e, Wgc[l], bgc[l].reshape(1, D),
                      Wbi[l], bbi[l].reshape(1, D))
        pieces.append(en)
    return pieces


def _fuse_run(base, wr, smalls, idx, sig_bs):
    n, m = base.shape[0], smalls[0].shape[0]
    group = NC * NS * FCH * 2
    n_pad = -(-n // group) * group
    zspread = 512
    m_pad = m + zspread
    fuse = _make_fuse(n_pad, m_pad)

    ratio = _make_sigmoid(n, sig_bs)(wr)

    rowids = jnp.arange(n, dtype=jnp.int32)
    inv = jnp.full((n,), -1, jnp.int32).at[idx].max(
        jnp.arange(idx.shape[0], dtype=jnp.int32))
    cidx = jnp.where(inv >= 0, inv, m + (rowids & (zspread - 1)))

    pad_n = n_pad - n
    base_p = jnp.concatenate([base, jnp.zeros((pad_n, FINAL), jnp.float32)])
    wr_p = jnp.concatenate([ratio, jnp.zeros((pad_n, FINAL), jnp.float32)])
    cidx_p = jnp.concatenate([cidx, jnp.full((pad_n,), m, jnp.int32)])
    sm_p = [jnp.concatenate([s, jnp.zeros((m_pad - m, D), jnp.float32)])
            for s in smalls]
    return fuse(base_p, wr_p, sm_p[0], sm_p[1], sm_p[2], cidx_p)[:n]


def kernel(edge_index0, values0, edge_index1, values1, idx_u, idx_i,
           user_emb0, item_emb0, user_emb1, item_emb1,
           Wgc0, bgc0, Wbi0, bbi0, Wgc1, bgc1, Wbi1, bbi1,
           W_ratio_u, W_ratio_i):
    nu0, ni0 = user_emb0.shape[0], item_emb0.shape[0]
    nu1 = user_emb1.shape[0]

    ego0 = jnp.concatenate([user_emb0, item_emb0], axis=0)
    ego1 = jnp.concatenate([user_emb1, item_emb1], axis=0)

    p0 = _ngcf_run(ego0, edge_index0[0], edge_index0[1], values0,
                   Wgc0, bgc0, Wbi0, bbi0, bs=1000)
    p1 = _ngcf_run(ego1, edge_index1[0], edge_index1[1], values1,
                   Wgc1, bgc1, Wbi1, bbi1, bs=1000)
    A0 = jnp.concatenate(p0, axis=1)

    final_u = _fuse_run(A0[:nu0], W_ratio_u, [p[:nu1] for p in p1],
                        idx_u, sig_bs=1000)
    final_i = _fuse_run(A0[nu0:], W_ratio_i, [p[nu1:] for p in p1],
                        idx_i, sig_bs=1000)
    return (final_u, final_i)


# speedup vs baseline: 1.3231x; 1.0386x over previous
import functools

import jax
import jax.numpy as jnp
from jax import lax
from jax.experimental import pallas as pl
from jax.experimental.pallas import tpu as pltpu
from jax.experimental.pallas import tpu_sc as plsc

D = 64
FINAL = 192
LANES = 16
NC = 2
NS = 16
ECHUNK = 80
FCH = 80


def _leaky(x):
    return jnp.where(x >= 0, x, 0.01 * x)


@functools.cache
def _make_spmm(n_nodes: int, n_edges: int):
    half = n_nodes // 2
    assert half * 2 == n_nodes
    e_per_tile = n_edges // NS
    G = e_per_tile // ECHUNK
    assert G * ECHUNK == e_per_tile and G >= 4
    half_pad = -(-(half + 64) // (NS * 8)) * (NS * 8)
    zrows = half_pad // NS
    wb = -(-half // NS)
    wb = -(-wb // 8) * 8
    wb_last = half - wb * (NS - 1)
    assert wb_last > 0 and wb_last % 8 == 0

    mesh = plsc.VectorSubcoreMesh(core_axis_name="c", subcore_axis_name="s")

    @functools.partial(
        pl.kernel,
        out_type=jax.ShapeDtypeStruct((n_nodes, D), jnp.float32),
        mesh=mesh,
        scratch_types=[
            pltpu.VMEM((2, ECHUNK), jnp.int32),
            pltpu.VMEM((2, ECHUNK), jnp.int32),
            pltpu.VMEM((2, ECHUNK), jnp.float32),
            pltpu.VMEM((2, ECHUNK), jnp.int32),
            pltpu.VMEM((2, ECHUNK, D), jnp.float32),
            pltpu.VMEM_SHARED((half_pad, D), jnp.float32),
            pltpu.SemaphoreType.DMA,
            pltpu.SemaphoreType.DMA,
            pltpu.SemaphoreType.DMA,
            pltpu.SemaphoreType.DMA,
            pltpu.SemaphoreType.DMA,
        ],
        compiler_params=pltpu.CompilerParams(use_tc_tiling_on_sc=False),
    )
    def spmm(ego_hbm, src_hbm, dst_hbm, vals_hbm, zeros_hbm, out_hbm,
             src_v, dst_v, vals_v, sidx_v, rows_v, acc, isem, gs0, gs1,
             ss0, ss1):
        cid = lax.axis_index("c")
        sid = lax.axis_index("s")
        off = cid * half
        gsem = (gs0, gs1)
        ssem = (ss0, ss1)

        pltpu.sync_copy(zeros_hbm.at[pl.ds(sid * zrows, zrows)],
                        acc.at[pl.ds(sid * zrows, zrows)])
        plsc.subcore_barrier()

        ebase = sid * e_per_tile

        def fire_idx(gg, b):
            eb = ebase + gg * ECHUNK
            pltpu.async_copy(src_hbm.at[pl.ds(eb, ECHUNK)], src_v.at[b], isem)
            pltpu.async_copy(dst_hbm.at[pl.ds(eb, ECHUNK)], dst_v.at[b], isem)
            pltpu.async_copy(vals_hbm.at[pl.ds(eb, ECHUNK)], vals_v.at[b], isem)

        def wait_idx(b):
            pltpu.make_async_copy(src_hbm.at[pl.ds(0, ECHUNK)], src_v.at[b],
                                  isem).wait()
            pltpu.make_async_copy(dst_hbm.at[pl.ds(0, ECHUNK)], dst_v.at[b],
                                  isem).wait()
            pltpu.make_async_copy(vals_hbm.at[pl.ds(0, ECHUNK)], vals_v.at[b],
                                  isem).wait()

        def fire_gather(b):
            pltpu.async_copy(ego_hbm.at[src_v.at[b]], rows_v.at[b], gsem[b])

        def wait_gather(b):
            pltpu.make_async_copy(ego_hbm.at[src_v.at[b]], rows_v.at[b],
                                  gsem[b]).wait()

        def fire_scatter(b):
            pltpu.async_copy(rows_v.at[b], acc.at[sidx_v.at[b]], ssem[b],
                             add=True)

        def wait_scatter(b):
            pltpu.make_async_copy(rows_v.at[b], acc.at[sidx_v.at[b]],
                                  ssem[b]).wait()

        def compute(b):
            for q in range(ECHUNK // LANES):
                sl = pl.ds(q * LANES, LANES)
                dvec = dst_v[b, sl]
                inr = (dvec >= off) & (dvec < off + half)
                sidx_v[b, sl] = jnp.where(inr, dvec - off,
                                          half + (dvec & 63))
                vvec = vals_v[b, sl]
                for j in range(LANES):
                    e = q * LANES + j
                    v = vvec[j]
                    for c in range(D // LANES):
                        cl = pl.ds(c * LANES, LANES)
                        rows_v[b, e, cl] = rows_v[b, e, cl] * v

        fire_idx(0, 0)
        wait_idx(0)
        fire_gather(0)
        fire_idx(1, 1)

        npairs = G // 2

        @pl.loop(0, npairs)
        def _pair(h):
            for b in (0, 1):
                gg = 2 * h + b
                nb = 1 - b

                @pl.when(gg + 1 < G)
                def _():
                    wait_idx(nb)

                @pl.when(gg >= 1)
                def _():
                    wait_scatter(nb)

                @pl.when(gg + 1 < G)
                def _():
                    fire_gather(nb)

                wait_gather(b)
                compute(b)
                fire_scatter(b)

                @pl.when(gg + 2 < G)
                def _():
                    fire_idx(gg + 2, b)

        if G % 2:
            wait_scatter(1)
            wait_gather(0)
            compute(0)
            fire_scatter(0)
            wait_scatter(0)
        else:
            wait_scatter(1)

        plsc.subcore_barrier()

        @pl.when(sid < NS - 1)
        def _wb_main():
            pltpu.sync_copy(acc.at[pl.ds(sid * wb, wb)],
                            out_hbm.at[pl.ds(off + sid * wb, wb)])

        @pl.when(sid == NS - 1)
        def _wb_tail():
            pltpu.sync_copy(acc.at[pl.ds((NS - 1) * wb, wb_last)],
                            out_hbm.at[pl.ds(off + (NS - 1) * wb, wb_last)])

    return spmm, half_pad


@functools.cache
def _make_dense(n_nodes: int, bs: int):
    grid = n_nodes // bs
    assert grid * bs == n_nodes

    def body(side_ref, ego_ref, wg_ref, bg_ref, wb_ref, bb_ref,
             enew_ref, enorm_ref):
        side = side_ref[...]
        ego = ego_ref[...]
        s = _leaky(jnp.dot(side, wg_ref[...],
                           preferred_element_type=jnp.float32) + bg_ref[...])
        b = _leaky(jnp.dot(ego * side, wb_ref[...],
                           preferred_element_type=jnp.float32) + bb_ref[...])
        e = s + b
        nrm = jnp.maximum(jnp.sqrt(jnp.sum(e * e, axis=1, keepdims=True)), 1e-12)
        enew_ref[...] = e
        enorm_ref[...] = e / nrm

    return pl.pallas_call(
        body,
        grid=(grid,),
        in_specs=[
            pl.BlockSpec((bs, D), lambda i: (i, 0)),
            pl.BlockSpec((bs, D), lambda i: (i, 0)),
            pl.BlockSpec((D, D), lambda i: (0, 0)),
            pl.BlockSpec((1, D), lambda i: (0, 0)),
            pl.BlockSpec((D, D), lambda i: (0, 0)),
            pl.BlockSpec((1, D), lambda i: (0, 0)),
        ],
        out_specs=[pl.BlockSpec((bs, D), lambda i: (i, 0)),
                   pl.BlockSpec((bs, D), lambda i: (i, 0))],
        out_shape=[jax.ShapeDtypeStruct((n_nodes, D), jnp.float32),
                   jax.ShapeDtypeStruct((n_nodes, D), jnp.float32)],
    )


@functools.cache
def _make_sigmoid(n_rows: int, bs: int):
    grid = n_rows // bs
    assert grid * bs == n_rows

    def body(w_ref, r_ref):
        r_ref[...] = jax.nn.sigmoid(w_ref[...])

    return pl.pallas_call(
        body,
        grid=(grid,),
        in_specs=[pl.BlockSpec((bs, FINAL), lambda i: (i, 0))],
        out_specs=pl.BlockSpec((bs, FINAL), lambda i: (i, 0)),
        out_shape=jax.ShapeDtypeStruct((n_rows, FINAL), jnp.float32),
    )


@functools.cache
def _make_fuse(n_pad: int, m_pad: int):
    per_worker = n_pad // (NC * NS)
    G = per_worker // FCH
    assert G * FCH == per_worker and G % 2 == 0 and G >= 4

    mesh = plsc.VectorSubcoreMesh(core_axis_name="c", subcore_axis_name="s")

    @functools.partial(
        pl.kernel,
        out_type=jax.ShapeDtypeStruct((n_pad, FINAL), jnp.float32),
        mesh=mesh,
        scratch_types=[
            pltpu.VMEM((2, FCH), jnp.int32),
            pltpu.VMEM((2, FCH, FINAL), jnp.float32),
            pltpu.VMEM((2, FCH, FINAL), jnp.float32),
            pltpu.VMEM((2, FCH, D), jnp.float32),
            pltpu.VMEM((2, FCH, D), jnp.float32),
            pltpu.VMEM((2, FCH, D), jnp.float32),
            pltpu.SemaphoreType.DMA,
            pltpu.SemaphoreType.DMA,
            pltpu.SemaphoreType.DMA,
            pltpu.SemaphoreType.DMA,
            pltpu.SemaphoreType.DMA,
        ],
        compiler_params=pltpu.CompilerParams(use_tc_tiling_on_sc=False),
    )
    def fuse(base_hbm, wr_hbm, sm0_hbm, sm1_hbm, sm2_hbm, cidx_hbm, out_hbm,
             idx_v, base_v, wr_v, g0_v, g1_v, g2_v, isem, gs0, gs1, ss0, ss1):
        cid = lax.axis_index("c")
        sid = lax.axis_index("s")
        wid = sid * NC + cid
        rbase = wid * per_worker
        gsem = (gs0, gs1)
        ssem = (ss0, ss1)

        def fire_loads(gg, b):
            rb = rbase + gg * FCH
            pltpu.async_copy(cidx_hbm.at[pl.ds(rb, FCH)], idx_v.at[b], isem)
            pltpu.async_copy(base_hbm.at[pl.ds(rb, FCH)], base_v.at[b], isem)
            pltpu.async_copy(wr_hbm.at[pl.ds(rb, FCH)], wr_v.at[b], isem)

        def wait_loads(b):
            pltpu.make_async_copy(cidx_hbm.at[pl.ds(0, FCH)], idx_v.at[b],
                                  isem).wait()
            pltpu.make_async_copy(base_hbm.at[pl.ds(0, FCH)], base_v.at[b],
                                  isem).wait()
            pltpu.make_async_copy(wr_hbm.at[pl.ds(0, FCH)], wr_v.at[b],
                                  isem).wait()

        smalls = (sm0_hbm, sm1_hbm, sm2_hbm)
        gaths = (g0_v, g1_v, g2_v)

        def fire_gather(b):
            for p in range(3):
                pltpu.async_copy(smalls[p].at[idx_v.at[b]], gaths[p].at[b],
                                 gsem[b])

        def wait_gather(b):
            for p in range(3):
                pltpu.make_async_copy(smalls[p].at[idx_v.at[b]],
                                      gaths[p].at[b], gsem[b]).wait()

        def fire_store(gg, b):
            rb = rbase + gg * FCH
            pltpu.async_copy(wr_v.at[b], out_hbm.at[pl.ds(rb, FCH)], ssem[b])

        def wait_store(b):
            pltpu.make_async_copy(wr_v.at[b], out_hbm.at[pl.ds(0, FCH)],
                                  ssem[b]).wait()

        def compute(b):
            def row(r, carry):
                for c in range(FINAL // LANES):
                    sl = pl.ds(c * LANES, LANES)
                    g = gaths[c // 4][b, r, pl.ds((c % 4) * LANES, LANES)]
                    wr_v[b, r, sl] = g + wr_v[b, r, sl] * (base_v[b, r, sl] - g)
                return carry
            lax.fori_loop(0, FCH, row, 0)

        fire_loads(0, 0)
        wait_loads(0)
        fire_gather(0)
        fire_loads(1, 1)

        @pl.loop(0, G // 2)
        def _pair(h):
            for b in (0, 1):
                gg = 2 * h + b
                nb = 1 - b

                @pl.when(gg + 1 < G)
                def _():
                    wait_loads(nb)

                @pl.when(gg >= 1)
                def _():
                    wait_store(nb)

                @pl.when(gg + 1 < G)
                def _():
                    fire_gather(nb)

                wait_gather(b)
                compute(b)
                fire_store(gg, b)

                @pl.when(gg + 2 < G)
                def _():
                    fire_loads(gg + 2, b)

        wait_store(1)

    return fuse


def _ngcf_run(ego, src, dst, vals, Wgc, bgc, Wbi, bbi, bs):
    n = ego.shape[0]
    spmm, half_pad = _make_spmm(n, src.shape[0])
    dense = _make_dense(n, bs)
    zeros = jnp.zeros((half_pad, D), jnp.float32)
    pieces = [ego]
    e = ego
    for l in range(Wgc.shape[0]):
        side = spmm(e, src, dst, vals, zeros)
        e, en = dense(side, e, Wgc[l], bgc[l].reshape(1, D),
                      Wbi[l], bbi[l].reshape(1, D))
        pieces.append(en)
    return pieces


def _fuse_run(base, wr, smalls, idx, sig_bs):
    n, m = base.shape[0], smalls[0].shape[0]
    group = NC * NS * FCH * 2
    n_pad = -(-n // group) * group
    zspread = 512
    m_pad = m + zspread
    fuse = _make_fuse(n_pad, m_pad)

    ratio = _make_sigmoid(n, sig_bs)(wr)

    rowids = jnp.arange(n, dtype=jnp.int32)
    inv = jnp.full((n,), -1, jnp.int32).at[idx].max(
        jnp.arange(idx.shape[0], dtype=jnp.int32))
    cidx = jnp.where(inv >= 0, inv, m + (rowids & (zspread - 1)))

    pad_n = n_pad - n
    base_p = jnp.concatenate([base, jnp.zeros((pad_n, FINAL), jnp.float32)])
    wr_p = jnp.concatenate([ratio, jnp.zeros((pad_n, FINAL), jnp.float32)])
    cidx_p = jnp.concatenate([cidx, jnp.full((pad_n,), m, jnp.int32)])
    sm_p = [jnp.concatenate([s, jnp.zeros((m_pad - m, D), jnp.float32)])
            for s in smalls]
    return fuse(base_p, wr_p, sm_p[0], sm_p[1], sm_p[2], cidx_p)[:n]


def kernel(edge_index0, values0, edge_index1, values1, idx_u, idx_i,
           user_emb0, item_emb0, user_emb1, item_emb1,
           Wgc0, bgc0, Wbi0, bbi0, Wgc1, bgc1, Wbi1, bbi1,
           W_ratio_u, W_ratio_i):
    nu0, ni0 = user_emb0.shape[0], item_emb0.shape[0]
    nu1 = user_emb1.shape[0]

    ego0 = jnp.concatenate([user_emb0, item_emb0], axis=0)
    ego1 = jnp.concatenate([user_emb1, item_emb1], axis=0)

    p0 = _ngcf_run(ego0, edge_index0[0], edge_index0[1], values0,
                   Wgc0, bgc0, Wbi0, bbi0, bs=1000)
    p1 = _ngcf_run(ego1, edge_index1[0], edge_index1[1], values1,
                   Wgc1, bgc1, Wbi1, bbi1, bs=1000)
    A0 = jnp.concatenate(p0, axis=1)

    final_u = _fuse_run(A0[:nu0], W_ratio_u, [p[:nu1] for p in p1],
                        idx_u, sig_bs=1000)
    final_i = _fuse_run(A0[nu0:], W_ratio_i, [p[nu1:] for p in p1],
                        idx_i, sig_bs=1000)
    return (final_u, final_i)
